# Initial kernel scaffold; baseline (speedup 1.0000x reference)
import jax
import jax.numpy as jnp
from jax.experimental import pallas as pl


def kernel(up, left, right):
    raise NotImplementedError("write your pallas kernel here")


# speedup vs baseline: 18.5495x; 18.5495x over previous
import functools

import jax
import jax.numpy as jnp
from jax import lax
from jax.experimental import pallas as pl
from jax.experimental.pallas import tpu as pltpu
from jax.experimental.pallas import tpu_sc as plsc

B, H, W = 64, 256, 256
NI = 110
NIP = 112
THRESH = 0.2
LMIN = 0.0235
NC, NS, L = 2, 16, 16
NW = NC * NS
BPW = B // NW
UCW = 4096
NUC = (H * W) // UCW


def _sc_loss(up3, left3, vals):
    mesh = plsc.VectorSubcoreMesh(core_axis_name="c", subcore_axis_name="s")

    @functools.partial(
        pl.kernel,
        out_type=jax.ShapeDtypeStruct((NW, L), jnp.float32),
        mesh=mesh,
        compiler_params=pltpu.CompilerParams(needs_layout_passes=False),
        scratch_types=[
            pltpu.VMEM((H * W,), jnp.float32),
            pltpu.VMEM((L * W,), jnp.float32),
            pltpu.VMEM((UCW,), jnp.float32),
            pltpu.VMEM((L,), jnp.float32),
            pltpu.VMEM((NIP,), jnp.float32),
        ],
    )
    def run(up_hbm, left_hbm, vals_hbm, out_hbm, s_ref, l_ref, up_ref, acc_ref, vals_ref):
        cid = lax.axis_index("c")
        sid = lax.axis_index("s")
        wid = sid * NC + cid
        lane = lax.iota(jnp.int32, L)
        acc = jnp.zeros((L,), jnp.float32)
        pltpu.sync_copy(vals_hbm, vals_ref)

        for t in range(BPW):
            b = wid * BPW + t

            def zbody(k, carry):
                s_ref[pl.ds(k * L, L)] = jnp.zeros((L,), jnp.float32)
                return carry

            lax.fori_loop(0, (H * W) // L, zbody, 0)

            def jbody(jc, carry):
                pltpu.sync_copy(left_hbm.at[b, jc], l_ref)
                jv = jc * L + lane
                lbase = lane * W

                def ibody(i, c2):
                    ii = jnp.full((L,), i, jnp.int32)
                    lv = plsc.load_gather(l_ref, [lbase + i])
                    x = jnp.float32(128.0) - lv * jnp.float32(60.0)
                    xp = x + jnp.float32(0.5)
                    y = xp.astype(jnp.int32)
                    tie = y.astype(jnp.float32) == xp
                    odd = (y & 1) == 1
                    r = y - jnp.where(tie & odd, 1, 0)
                    r = jnp.clip(r, 0, H - 1)
                    flat = r * W + jv
                    g = plsc.load_gather(s_ref, [flat])
                    wmask = (lv >= jnp.float32(LMIN)) & (g == jnp.float32(0.0))
                    vv = plsc.load_gather(vals_ref, [ii])
                    plsc.store_scatter(s_ref, [flat], vv, mask=wmask)
                    return c2

                lax.fori_loop(0, NI, ibody, 0)
                return carry

            lax.fori_loop(0, H // L, jbody, 0)

            def ubody(u, a):
                pltpu.sync_copy(up_hbm.at[b, u], up_ref)

                def kbody(k, a2):
                    sv = s_ref[pl.ds(u * UCW + k * L, L)]
                    uv = up_ref[pl.ds(k * L, L)]
                    d = jnp.abs(sv - uv)
                    return a2 + jnp.where(d < jnp.float32(THRESH), d, jnp.float32(0.0))

                return lax.fori_loop(0, UCW // L, kbody, a)

            acc = lax.fori_loop(0, NUC, ubody, acc)

        acc_ref[...] = acc
        pltpu.sync_copy(acc_ref, out_hbm.at[wid])

    return run(up3, left3, vals)


@jax.jit
def kernel(up, left, right):
    del right
    up3 = up.reshape(B, NUC, UCW)
    left3 = left.reshape(B, H // L, L * W)
    ivec = jnp.arange(NIP, dtype=jnp.float32)
    vals = (jnp.float32(110.0) - ivec) / jnp.float32(50.0)
    partials = _sc_loss(up3, left3, vals)
    return jnp.sum(partials) / jnp.float32(B * H * W)


# speedup vs baseline: 37.4356x; 2.0181x over previous
import functools

import jax
import jax.numpy as jnp
from jax import lax
from jax.experimental import pallas as pl
from jax.experimental.pallas import tpu as pltpu
from jax.experimental.pallas import tpu_sc as plsc

B, H, W = 64, 256, 256
NI = 110
NIP = 112
THRESH = 0.2
LMIN = 0.0235
NC, NS, L = 2, 16, 16
NW = NC * NS
BPW = B // NW
NJC = W // L
UCW = 16384
NUC = (H * W) // UCW


def _sc_loss(up3, left2, vals):
    mesh = plsc.VectorSubcoreMesh(core_axis_name="c", subcore_axis_name="s")

    @functools.partial(
        pl.kernel,
        out_type=jax.ShapeDtypeStruct((NW, L), jnp.float32),
        mesh=mesh,
        compiler_params=pltpu.CompilerParams(needs_layout_passes=False),
        scratch_types=[
            pltpu.VMEM((H * W,), jnp.float32),
            pltpu.VMEM((NI * W,), jnp.float32),
            pltpu.VMEM((UCW,), jnp.float32),
            pltpu.VMEM((UCW,), jnp.float32),
            pltpu.VMEM((NIP,), jnp.float32),
            pltpu.VMEM((L,), jnp.float32),
            pltpu.SemaphoreType.DMA,
            pltpu.SemaphoreType.DMA,
            pltpu.SemaphoreType.DMA,
        ],
    )
    def run(up_hbm, left_hbm, vals_hbm, out_hbm, s_ref, l_ref, upa, upb, vals_ref, acc_ref, sema, semb, seml):
        cid = lax.axis_index("c")
        sid = lax.axis_index("s")
        wid = sid * NC + cid
        lane = lax.iota(jnp.int32, L)
        jvs = [jc * L + lane for jc in range(NJC)]
        zero16 = jnp.zeros((L,), jnp.float32)

        pltpu.sync_copy(vals_hbm, vals_ref)
        b0 = wid * BPW
        pltpu.async_copy(left_hbm.at[b0], l_ref, seml)

        def zbody(k, carry):
            base = k * (8 * L)
            for q in range(8):
                s_ref[pl.ds(base + q * L, L)] = zero16
            return carry

        lax.fori_loop(0, (H * W) // (8 * L), zbody, 0)
        pltpu.make_async_copy(left_hbm.at[b0], l_ref, seml).wait()

        accs = (zero16, zero16, zero16, zero16)
        for t in range(BPW):
            b = b0 + t

            pltpu.async_copy(up_hbm.at[b, 0], upa, sema)

            def ibody(i, carry):
                vv = plsc.load_gather(vals_ref, [jnp.full((L,), i, jnp.int32)])
                ibase = i * W
                for jc in range(NJC):
                    lv = l_ref[pl.ds(ibase + jc * L, L)]
                    x = jnp.float32(128.0) - lv * jnp.float32(60.0)
                    xp = x + jnp.float32(0.5)
                    y = xp.astype(jnp.int32)
                    tie = y.astype(jnp.float32) == xp
                    odd = (y & 1) == 1
                    r = y - jnp.where(tie & odd, 1, 0)
                    r = jnp.clip(r, 0, H - 1)
                    flat = r * W + jvs[jc]
                    g = plsc.load_gather(s_ref, [flat])
                    wm = (lv >= jnp.float32(LMIN)) & (g == jnp.float32(0.0))
                    plsc.store_scatter(s_ref, [flat], vv, mask=wm)
                return carry

            lax.fori_loop(0, NI, ibody, 0)

            if t + 1 < BPW:
                pltpu.async_copy(left_hbm.at[b + 1], l_ref, seml)

            for c in range(NUC):
                cur, sem = (upa, sema) if c % 2 == 0 else (upb, semb)
                if c + 1 < NUC:
                    nxt, nsem = (upb, semb) if c % 2 == 0 else (upa, sema)
                    pltpu.async_copy(up_hbm.at[b, c + 1], nxt, nsem)
                pltpu.make_async_copy(up_hbm.at[b, c], cur, sem).wait()
                choff = c * UCW

                def kbody(k, a4, cur=cur, choff=choff):
                    base = k * (4 * L)
                    outs = []
                    for q in range(4):
                        off = base + q * L
                        sv = s_ref[pl.ds(choff + off, L)]
                        uv = cur[pl.ds(off, L)]
                        d = jnp.abs(sv - uv)
                        outs.append(a4[q] + jnp.where(d < jnp.float32(THRESH), d, jnp.float32(0.0)))
                        s_ref[pl.ds(choff + off, L)] = zero16
                    return tuple(outs)

                accs = lax.fori_loop(0, UCW // (4 * L), kbody, accs)

            if t + 1 < BPW:
                pltpu.make_async_copy(left_hbm.at[b + 1], l_ref, seml).wait()

        acc_ref[...] = (accs[0] + accs[1]) + (accs[2] + accs[3])
        pltpu.sync_copy(acc_ref, out_hbm.at[wid])

    return run(up3, left2, vals)


@jax.jit
def kernel(up, left, right):
    del right
    up3 = up.reshape(B, NUC, UCW)
    left2 = jnp.transpose(left.reshape(B, H, W)[:, :, :NI], (0, 2, 1)).reshape(B, NI * W)
    ivec = jnp.arange(NIP, dtype=jnp.float32)
    vals = (jnp.float32(110.0) - ivec) / jnp.float32(50.0)
    partials = _sc_loss(up3, left2, vals)
    return jnp.sum(partials) / jnp.float32(B * H * W)
